# Initial kernel scaffold; baseline (speedup 1.0000x reference)
import jax
import jax.numpy as jnp
from jax.experimental import pallas as pl


def kernel(node_features, edge_features, edge_indices, W_node, b_node, W_edge, b_edge, W_out, b_out):
    raise NotImplementedError("write your pallas kernel here")


# speedup vs baseline: 2.3919x; 2.3919x over previous
import functools

import jax
import jax.numpy as jnp
from jax import lax
from jax.experimental import pallas as pl
from jax.experimental.pallas import tpu as pltpu
from jax.experimental.pallas import tpu_sc as plsc

N_NODES = 10000
N_EDGES = 320000
D = 128
NC, NS, L = 2, 16, 16
NW = NC * NS
EPW = N_EDGES // NW
K = 80
NCHUNK = EPW // K
STRIPE = 640
TAIL = N_NODES - 15 * STRIPE
ZR = 80


def _mm_bias_body(x_ref, w_ref, b_ref, o_ref):
    o_ref[...] = (
        jnp.dot(x_ref[...], w_ref[...], preferred_element_type=jnp.float32)
        + b_ref[...]
    )


def _mm_bias(x, w, b, block_rows):
    m, k = x.shape
    n = w.shape[1]
    return pl.pallas_call(
        _mm_bias_body,
        grid=(m // block_rows,),
        in_specs=[
            pl.BlockSpec((block_rows, k), lambda i: (i, 0)),
            pl.BlockSpec((k, n), lambda i: (0, 0)),
            pl.BlockSpec((1, n), lambda i: (0, 0)),
        ],
        out_specs=pl.BlockSpec((block_rows, n), lambda i: (i, 0)),
        out_shape=jax.ShapeDtypeStruct((m, n), jnp.float32),
    )(x, w, b.reshape(1, n))


def _final_body(p_ref, w_ref, b_ref, o_ref):
    a = p_ref[0] + p_ref[1]
    o_ref[...] = (
        jnp.dot(a, w_ref[...], preferred_element_type=jnp.float32) + b_ref[...]
    )


def _final_mm(partials, w, b, block_rows):
    _, m, n = partials.shape
    return pl.pallas_call(
        _final_body,
        grid=(m // block_rows,),
        in_specs=[
            pl.BlockSpec((2, block_rows, n), lambda i: (0, i, 0)),
            pl.BlockSpec((n, n), lambda i: (0, 0)),
            pl.BlockSpec((1, n), lambda i: (0, 0)),
        ],
        out_specs=pl.BlockSpec((block_rows, n), lambda i: (i, 0)),
        out_shape=jax.ShapeDtypeStruct((m, n), jnp.float32),
    )(partials, w, b.reshape(1, n))


_mesh = plsc.VectorSubcoreMesh(core_axis_name="c", subcore_axis_name="s")


@functools.partial(
    pl.kernel,
    out_type=jax.ShapeDtypeStruct((NC, N_NODES, D), jnp.float32),
    mesh=_mesh,
    scratch_types=[
        pltpu.VMEM((K,), jnp.int32),
        pltpu.VMEM((K,), jnp.int32),
        pltpu.VMEM((K, D), jnp.float32),
        pltpu.VMEM((K, D), jnp.float32),
        pltpu.VMEM((ZR, D), jnp.float32),
        pltpu.VMEM_SHARED((N_NODES, D), jnp.float32),
        pltpu.SemaphoreType.DMA,
    ],
)
def _sc_gather_mul_scatter(
    node_t, edge_t, src, dst, out,
    src_v, dst_v, gat_v, edg_v, zero_v, agg_sh, gsem,
):
    c = lax.axis_index("c")
    s = lax.axis_index("s")
    wid = s * NC + c

    zvec = jnp.zeros((L,), jnp.float32)

    def zrow(r, carry):
        for v in range(D // L):
            zero_v[r, pl.ds(v * L, L)] = zvec
        return carry

    lax.fori_loop(0, ZR, zrow, 0)
    base = s * STRIPE
    for z in range(TAIL // ZR):
        pltpu.sync_copy(zero_v, agg_sh.at[pl.ds(base + z * ZR, ZR)])

    @pl.when(s < NS - 1)
    def _zero_rest():
        for z in range(TAIL // ZR, STRIPE // ZR):
            pltpu.sync_copy(zero_v, agg_sh.at[pl.ds(base + z * ZR, ZR)])

    plsc.subcore_barrier()

    def chunk(g, carry):
        eoff = wid * EPW + g * K
        pltpu.sync_copy(src.at[pl.ds(eoff, K)], src_v)
        pltpu.sync_copy(dst.at[pl.ds(eoff, K)], dst_v)
        pltpu.sync_copy(edge_t.at[pl.ds(eoff, K)], edg_v)
        pltpu.async_copy(node_t.at[src_v], gat_v, gsem).wait()

        def mul(e, inner):
            for v in range(D // L):
                sl = pl.ds(v * L, L)
                gat_v[e, sl] = gat_v[e, sl] * edg_v[e, sl]
            return inner

        lax.fori_loop(0, K, mul, 0)
        pltpu.sync_copy(gat_v, agg_sh.at[dst_v], add=True)
        return carry

    lax.fori_loop(0, NCHUNK, chunk, 0)

    plsc.subcore_barrier()
    pltpu.sync_copy(
        agg_sh.at[pl.ds(base, TAIL)],
        out.at[c, pl.ds(base, TAIL)],
    )

    @pl.when(s < NS - 1)
    def _write_rest():
        pltpu.sync_copy(
            agg_sh.at[pl.ds(base + TAIL, STRIPE - TAIL)],
            out.at[c, pl.ds(base + TAIL, STRIPE - TAIL)],
        )


def kernel(node_features, edge_features, edge_indices,
           W_node, b_node, W_edge, b_edge, W_out, b_out):
    node_t = _mm_bias(node_features, W_node, b_node, 1000)
    edge_t = _mm_bias(edge_features, W_edge, b_edge, 2000)
    ei = edge_indices.astype(jnp.int32)
    src = ei[:, 0]
    dst = ei[:, 1]
    partials = _sc_gather_mul_scatter(node_t, edge_t, src, dst)
    return _final_mm(partials, W_out, b_out, 1000)


# speedup vs baseline: 3.4971x; 1.4621x over previous
import functools

import jax
import jax.numpy as jnp
from jax import lax
from jax.experimental import pallas as pl
from jax.experimental.pallas import tpu as pltpu
from jax.experimental.pallas import tpu_sc as plsc

N_NODES = 10000
N_EDGES = 320000
D = 128
NC, NS, L = 2, 16, 16
NW = NC * NS
EPW = N_EDGES // NW
K = 40
NCHUNK = EPW // K
NSEG = 5
SEG = NCHUNK // NSEG
STRIPE = 640
TAIL = N_NODES - 15 * STRIPE


def _mm_bias_body(x_ref, w_ref, b_ref, o_ref):
    o_ref[...] = (
        jnp.dot(x_ref[...], w_ref[...], preferred_element_type=jnp.float32)
        + b_ref[...]
    )


def _mm_bias(x, w, b, block_rows):
    m, k = x.shape
    n = w.shape[1]
    return pl.pallas_call(
        _mm_bias_body,
        grid=(m // block_rows,),
        in_specs=[
            pl.BlockSpec((block_rows, k), lambda i: (i, 0)),
            pl.BlockSpec((k, n), lambda i: (0, 0)),
            pl.BlockSpec((1, n), lambda i: (0, 0)),
        ],
        out_specs=pl.BlockSpec((block_rows, n), lambda i: (i, 0)),
        out_shape=jax.ShapeDtypeStruct((m, n), jnp.float32),
    )(x, w, b.reshape(1, n))


def _final_body(p_ref, w_ref, b_ref, o_ref):
    a = p_ref[0] + p_ref[1]
    o_ref[...] = (
        jnp.dot(a, w_ref[...], preferred_element_type=jnp.float32) + b_ref[...]
    )


def _final_mm(partials, w, b, block_rows):
    _, m, n = partials.shape
    return pl.pallas_call(
        _final_body,
        grid=(m // block_rows,),
        in_specs=[
            pl.BlockSpec((2, block_rows, n), lambda i: (0, i, 0)),
            pl.BlockSpec((n, n), lambda i: (0, 0)),
            pl.BlockSpec((1, n), lambda i: (0, 0)),
        ],
        out_specs=pl.BlockSpec((block_rows, n), lambda i: (i, 0)),
        out_shape=jax.ShapeDtypeStruct((m, n), jnp.float32),
    )(partials, w, b.reshape(1, n))


_mesh = plsc.VectorSubcoreMesh(core_axis_name="c", subcore_axis_name="s")


@functools.partial(
    pl.kernel,
    out_type=jax.ShapeDtypeStruct((NC, N_NODES, D), jnp.float32),
    mesh=_mesh,
    scratch_types=[
        pltpu.VMEM((SEG, K), jnp.int32),
        pltpu.VMEM((SEG, K), jnp.int32),
        pltpu.VMEM((K, D), jnp.float32),
        pltpu.VMEM((K, D), jnp.float32),
        pltpu.VMEM((K, D), jnp.float32),
        pltpu.VMEM((K, D), jnp.float32),
        pltpu.VMEM_SHARED((N_NODES, D), jnp.float32),
        pltpu.SemaphoreType.DMA,
        pltpu.SemaphoreType.DMA,
        pltpu.SemaphoreType.DMA,
        pltpu.SemaphoreType.DMA,
        pltpu.SemaphoreType.DMA,
        pltpu.SemaphoreType.DMA,
    ],
)
def _sc_gather_mul_scatter(
    node_t, edge_t, src4, dst4, out,
    src_v, dst_v, gat0, gat1, edg0, edg1, agg_sh,
    gsem0, gsem1, esem0, esem1, ssem0, ssem1,
):
    c = lax.axis_index("c")
    s = lax.axis_index("s")
    wid = s * NC + c
    gat = (gat0, gat1)
    edg = (edg0, edg1)
    gsem = (gsem0, gsem1)
    esem = (esem0, esem1)
    ssem = (ssem0, ssem1)

    zvec = jnp.zeros((L,), jnp.float32)

    def zrow(r, carry):
        for v in range(D // L):
            gat0[r, pl.ds(v * L, L)] = zvec
        return carry

    lax.fori_loop(0, K, zrow, 0)
    base = s * STRIPE
    for z in range(TAIL // K):
        pltpu.sync_copy(gat0, agg_sh.at[pl.ds(base + z * K, K)])

    @pl.when(s < NS - 1)
    def _zero_rest():
        for z in range(TAIL // K, STRIPE // K):
            pltpu.sync_copy(gat0, agg_sh.at[pl.ds(base + z * K, K)])

    plsc.subcore_barrier()

    ebase = wid * EPW

    def seg_body(seg, carry):
        soff = seg * SEG
        pltpu.sync_copy(src4.at[wid, seg], src_v)
        pltpu.sync_copy(dst4.at[wid, seg], dst_v)

        def fetch(l, b):
            pltpu.async_copy(
                edge_t.at[pl.ds(ebase + (soff + l) * K, K)], edg[b], esem[b])
            pltpu.async_copy(node_t.at[src_v.at[l]], gat[b], gsem[b])

        def wait_fetch(l, b):
            pltpu.make_async_copy(
                edge_t.at[pl.ds(ebase + (soff + l) * K, K)], edg[b], esem[b]).wait()
            pltpu.make_async_copy(node_t.at[src_v.at[l]], gat[b], gsem[b]).wait()

        def multiply(b):
            ga, eb = gat[b], edg[b]

            def mul(e, inner):
                for v in range(D // L):
                    sl = pl.ds(v * L, L)
                    ga[e, sl] = ga[e, sl] * eb[e, sl]
                return inner

            lax.fori_loop(0, K, mul, 0)

        def scatter(l, b):
            pltpu.async_copy(gat[b], agg_sh.at[dst_v.at[l]], ssem[b], add=True)

        def wait_scatter(l, b):
            pltpu.make_async_copy(gat[b], agg_sh.at[dst_v.at[l]], ssem[b]).wait()

        fetch(0, 0)

        def pair(p, inner):
            la, lb = 2 * p, 2 * p + 1

            @pl.when(p > 0)
            def _ws1():
                wait_scatter(lb - 2, 1)

            fetch(lb, 1)
            wait_fetch(la, 0)
            multiply(0)
            scatter(la, 0)
            wait_fetch(lb, 1)
            multiply(1)
            wait_scatter(la, 0)

            @pl.when(p < SEG // 2 - 1)
            def _f0():
                fetch(la + 2, 0)

            scatter(lb, 1)
            return inner

        lax.fori_loop(0, SEG // 2, pair, 0)
        wait_scatter(SEG - 1, 1)
        return carry

    lax.fori_loop(0, NSEG, seg_body, 0)

    plsc.subcore_barrier()
    pltpu.sync_copy(
        agg_sh.at[pl.ds(base, TAIL)],
        out.at[c, pl.ds(base, TAIL)],
    )

    @pl.when(s < NS - 1)
    def _write_rest():
        pltpu.sync_copy(
            agg_sh.at[pl.ds(base + TAIL, STRIPE - TAIL)],
            out.at[c, pl.ds(base + TAIL, STRIPE - TAIL)],
        )


def kernel(node_features, edge_features, edge_indices,
           W_node, b_node, W_edge, b_edge, W_out, b_out):
    node_t = _mm_bias(node_features, W_node, b_node, 1000)
    edge_t = _mm_bias(edge_features, W_edge, b_edge, 2000)
    ei = edge_indices.astype(jnp.int32)
    src4 = ei[:, 0].reshape(NW, NSEG, SEG, K)
    dst4 = ei[:, 1].reshape(NW, NSEG, SEG, K)
    partials = _sc_gather_mul_scatter(node_t, edge_t, src4, dst4)
    return _final_mm(partials, W_out, b_out, 1000)
